# baseline (speedup 1.0000x reference)
import functools

import jax
import jax.numpy as jnp
from jax import lax
from jax.experimental import pallas as pl
from jax.experimental.pallas import tpu as pltpu
from jax.experimental.pallas import tpu_sc as plsc

N = 10000
E = 320000
H = 128
EPS = 1e-5

NC = 2
NS = 16
NT = NC * NS
CH = 96
NSUB = 112
EPT = NSUB * CH
EPAD = NT * EPT
NA = N + 16
RPS = 624


def _sc_scatter_body(src_hbm, dst_hbm, table_hbm, zeros_hbm, out_hbm,
                     src_v, dst_v, rows0, rows1, rows2, acc_sh, sem0, sem1,
                     sem2, ssem0, ssem1, ssem2):
    c = lax.axis_index("c")
    s = lax.axis_index("s")
    t = c * NS + s

    pltpu.sync_copy(zeros_hbm.at[pl.ds(s * RPS, RPS)],
                    acc_sh.at[pl.ds(s * RPS, RPS)])

    @pl.when(s == NS - 1)
    def _init_tail():
        pltpu.sync_copy(zeros_hbm.at[pl.ds(NS * RPS, NA - NS * RPS)],
                        acc_sh.at[pl.ds(NS * RPS, NA - NS * RPS)])

    plsc.subcore_barrier()

    for q0, qn in ((0, 32), (32, 32), (64, 32), (96, 16)):
        base = t * NSUB + q0
        pltpu.sync_copy(src_hbm.at[pl.ds(base, qn)], src_v.at[pl.ds(0, qn)])
        pltpu.sync_copy(dst_hbm.at[pl.ds(base, qn)], dst_v.at[pl.ds(0, qn)])

        @pl.loop(0, qn, step=4)
        def _chunk(j):
            def gat(k, buf, sem):
                return pltpu.async_copy(table_hbm.at[src_v.at[k]], buf, sem)

            def sca(k, buf, sem):
                return pltpu.async_copy(buf, acc_sh.at[dst_v.at[k]], sem,
                                        add=True)

            g0 = gat(j, rows0, sem0)
            g1 = gat(j + 1, rows1, sem1)
            g2 = gat(j + 2, rows2, sem2)
            g0.wait()
            s0 = sca(j, rows0, ssem0)
            g1.wait()
            s1 = sca(j + 1, rows1, ssem1)
            s0.wait()
            g3 = gat(j + 3, rows0, sem0)
            g2.wait()
            s2 = sca(j + 2, rows2, ssem2)
            g3.wait()
            s3 = sca(j + 3, rows0, ssem0)
            s1.wait()
            s2.wait()
            s3.wait()

    plsc.subcore_barrier()
    pltpu.sync_copy(acc_sh.at[pl.ds(s * RPS, RPS)],
                    out_hbm.at[c, pl.ds(s * RPS, RPS)])

    @pl.when(s == NS - 1)
    def _out_tail():
        pltpu.sync_copy(acc_sh.at[pl.ds(NS * RPS, N - NS * RPS)],
                        out_hbm.at[c, pl.ds(NS * RPS, N - NS * RPS)])


_sc_scatter = functools.partial(
    pl.kernel,
    out_type=jax.ShapeDtypeStruct((NC, N, H), jnp.float32),
    mesh=plsc.VectorSubcoreMesh(core_axis_name="c", subcore_axis_name="s"),
    scratch_types=[
        pltpu.VMEM((32, CH), jnp.int32),
        pltpu.VMEM((32, CH), jnp.int32),
        pltpu.VMEM((CH, H), jnp.float32),
        pltpu.VMEM((CH, H), jnp.float32),
        pltpu.VMEM((CH, H), jnp.float32),
        pltpu.VMEM_SHARED((NA, H), jnp.float32),
        pltpu.SemaphoreType.DMA,
        pltpu.SemaphoreType.DMA,
        pltpu.SemaphoreType.DMA,
        pltpu.SemaphoreType.DMA,
        pltpu.SemaphoreType.DMA,
        pltpu.SemaphoreType.DMA,
    ],
)(_sc_scatter_body)


def _style(h0, wv, W, b, aW, ab, ns, noise):
    h = jnp.dot(h0, W, preferred_element_type=jnp.float32) + b
    h = h + noise * ns
    sty = jnp.dot(wv, aW, preferred_element_type=jnp.float32) + ab
    gamma = sty[:, :H]
    beta = sty[:, H:]
    mu = jnp.mean(h, axis=0, keepdims=True)
    var = jnp.mean((h - mu) ** 2, axis=0, keepdims=True)
    xn = (h - mu) / jnp.sqrt(var + EPS)
    o = gamma * xn + beta
    return jnp.where(o >= 0, o, 0.01 * o)


def _xn_body(x_ref, w_ref, nm_W_ref, nm_b_ref, nm_aW_ref, nm_ab_ref,
             nm_ns_ref, nm_noise_ref, cat2_W_ref, cat2_b_ref, out_ref):
    xn = _style(x_ref[...], w_ref[...], nm_W_ref[...], nm_b_ref[...],
                nm_aW_ref[...], nm_ab_ref[...], nm_ns_ref[...],
                nm_noise_ref[...])
    out_ref[...] = xn + jnp.dot(xn, cat2_W_ref[...],
                                preferred_element_type=jnp.float32) + cat2_b_ref[...]


def _dense_body(parts_ref, xn2_ref, w_ref, edge_b_ref, cat1_W_ref, cat1_b_ref,
                f1_W_ref, f1_b_ref, f1_aW_ref, f1_ab_ref, f1_ns_ref, f1_noise_ref,
                f2_W_ref, f2_b_ref, f2_aW_ref, f2_ab_ref, f2_ns_ref, f2_noise_ref,
                out_ref):
    wv = w_ref[...]
    agg = parts_ref[0] + parts_ref[1] + edge_b_ref[...]
    out = agg + jnp.dot(agg, cat1_W_ref[...],
                        preferred_element_type=jnp.float32) + cat1_b_ref[...]
    out = jnp.maximum(out + xn2_ref[...], 0.0)
    out = _style(out, wv, f1_W_ref[...], f1_b_ref[...], f1_aW_ref[...],
                 f1_ab_ref[...], f1_ns_ref[...], f1_noise_ref[...])
    out = _style(out, wv, f2_W_ref[...], f2_b_ref[...], f2_aW_ref[...],
                 f2_ab_ref[...], f2_ns_ref[...], f2_noise_ref[...])
    out_ref[...] = out


def _dense(parts, x, w, edge_b, cat1_W, cat1_b, cat2_W, cat2_b,
           nm_W, nm_b, nm_aW, nm_ab, nm_ns, nm_noise,
           f1_W, f1_b, f1_aW, f1_ab, f1_ns, f1_noise,
           f2_W, f2_b, f2_aW, f2_ab, f2_ns, f2_noise):
    r2 = lambda v: v.reshape(1, -1)
    xn2 = pl.pallas_call(
        _xn_body,
        out_shape=jax.ShapeDtypeStruct((N, H), jnp.float32),
    )(x, w, nm_W, r2(nm_b), nm_aW, r2(nm_ab), r2(nm_ns), nm_noise,
      cat2_W, r2(cat2_b))
    return pl.pallas_call(
        _dense_body,
        out_shape=jax.ShapeDtypeStruct((N, H), jnp.float32),
    )(parts, xn2, w, r2(edge_b), cat1_W, r2(cat1_b),
      f1_W, r2(f1_b), f1_aW, r2(f1_ab), r2(f1_ns), f1_noise,
      f2_W, r2(f2_b), f2_aW, r2(f2_ab), r2(f2_ns), f2_noise)


def kernel(x, edge_index, w, edge_W, edge_b, cat1_W, cat1_b, cat2_W, cat2_b,
           nm_W, nm_b, nm_aW, nm_ab, nm_ns, nm_noise,
           f1_W, f1_b, f1_aW, f1_ab, f1_ns, f1_noise,
           f2_W, f2_b, f2_aW, f2_ab, f2_ns, f2_noise):
    src = edge_index[0]
    dst = edge_index[1]
    npad = EPAD - E
    ar = jnp.arange(npad, dtype=jnp.int32)
    src_p = jnp.concatenate([src, (ar * 131) % N]).reshape(NT * NSUB, CH)
    dst_p = jnp.concatenate([dst, N + (ar % 16)]).reshape(NT * NSUB, CH)
    zeros = jnp.zeros((NA, H), jnp.float32)
    parts = _sc_scatter(src_p, dst_p, edge_W, zeros)
    return _dense(parts, x, w, edge_b, cat1_W, cat1_b, cat2_W, cat2_b,
                  nm_W, nm_b, nm_aW, nm_ab, nm_ns, nm_noise,
                  f1_W, f1_b, f1_aW, f1_ab, f1_ns, f1_noise,
                  f2_W, f2_b, f2_aW, f2_ab, f2_ns, f2_noise)


# speedup vs baseline: 1.1151x; 1.1151x over previous
import functools

import jax
import jax.numpy as jnp
from jax import lax
from jax.experimental import pallas as pl
from jax.experimental.pallas import tpu as pltpu
from jax.experimental.pallas import tpu_sc as plsc

N = 10000
E = 320000
H = 128
EPS = 1e-5

NC = 2
NS = 16
NT = NC * NS
CH = 128
NSUB = 80
HALF = NSUB // 2
EPT = NSUB * CH
EPAD = NT * EPT
NA = N + 16
RPS = 624


def _sc_scatter_body(src_hbm, dst_hbm, table_hbm, zeros_hbm, out_hbm,
                     src_v, dst_v, rows0, rows1, acc_sh, sem0, sem1,
                     ssem0, ssem1):
    c = lax.axis_index("c")
    s = lax.axis_index("s")
    t = c * NS + s

    pltpu.sync_copy(zeros_hbm.at[pl.ds(s * RPS, RPS)],
                    acc_sh.at[pl.ds(s * RPS, RPS)])

    @pl.when(s == NS - 1)
    def _init_tail():
        pltpu.sync_copy(zeros_hbm.at[pl.ds(NS * RPS, NA - NS * RPS)],
                        acc_sh.at[pl.ds(NS * RPS, NA - NS * RPS)])

    plsc.subcore_barrier()

    for h in range(2):
        base = t * NSUB + h * HALF
        pltpu.sync_copy(src_hbm.at[pl.ds(base, HALF)], src_v)
        pltpu.sync_copy(dst_hbm.at[pl.ds(base, HALF)], dst_v)

        @pl.loop(0, HALF, step=4)
        def _chunk(j):
            def gat(k, buf, sem):
                return pltpu.async_copy(table_hbm.at[src_v.at[k]], buf, sem)

            def sca(k, buf, sem):
                return pltpu.async_copy(buf, acc_sh.at[dst_v.at[k]], sem,
                                        add=True)

            g0 = gat(j, rows0, sem0)
            g1 = gat(j + 1, rows1, sem1)
            g0.wait()
            s0 = sca(j, rows0, ssem0)
            g1.wait()
            s1 = sca(j + 1, rows1, ssem1)
            s0.wait()
            g2 = gat(j + 2, rows0, sem0)
            s1.wait()
            g3 = gat(j + 3, rows1, sem1)
            g2.wait()
            s2 = sca(j + 2, rows0, ssem0)
            g3.wait()
            s3 = sca(j + 3, rows1, ssem1)
            s2.wait()
            s3.wait()

    plsc.subcore_barrier()
    pltpu.sync_copy(acc_sh.at[pl.ds(s * RPS, RPS)],
                    out_hbm.at[c, pl.ds(s * RPS, RPS)])

    @pl.when(s == NS - 1)
    def _out_tail():
        pltpu.sync_copy(acc_sh.at[pl.ds(NS * RPS, N - NS * RPS)],
                        out_hbm.at[c, pl.ds(NS * RPS, N - NS * RPS)])


_sc_scatter = functools.partial(
    pl.kernel,
    out_type=jax.ShapeDtypeStruct((NC, N, H), jnp.float32),
    mesh=plsc.VectorSubcoreMesh(core_axis_name="c", subcore_axis_name="s"),
    scratch_types=[
        pltpu.VMEM((HALF, CH), jnp.int32),
        pltpu.VMEM((HALF, CH), jnp.int32),
        pltpu.VMEM((CH, H), jnp.float32),
        pltpu.VMEM((CH, H), jnp.float32),
        pltpu.VMEM_SHARED((NA, H), jnp.float32),
        pltpu.SemaphoreType.DMA,
        pltpu.SemaphoreType.DMA,
        pltpu.SemaphoreType.DMA,
        pltpu.SemaphoreType.DMA,
    ],
)(_sc_scatter_body)


def _style(h0, wv, W, b, aW, ab, ns, noise):
    h = jnp.dot(h0, W, preferred_element_type=jnp.float32) + b
    h = h + noise * ns
    sty = jnp.dot(wv, aW, preferred_element_type=jnp.float32) + ab
    gamma = sty[:, :H]
    beta = sty[:, H:]
    mu = jnp.mean(h, axis=0, keepdims=True)
    var = jnp.mean((h - mu) ** 2, axis=0, keepdims=True)
    xn = (h - mu) / jnp.sqrt(var + EPS)
    o = gamma * xn + beta
    return jnp.where(o >= 0, o, 0.01 * o)


def _xn_body(x_ref, w_ref, nm_W_ref, nm_b_ref, nm_aW_ref, nm_ab_ref,
             nm_ns_ref, nm_noise_ref, cat2_W_ref, cat2_b_ref, out_ref):
    xn = _style(x_ref[...], w_ref[...], nm_W_ref[...], nm_b_ref[...],
                nm_aW_ref[...], nm_ab_ref[...], nm_ns_ref[...],
                nm_noise_ref[...])
    out_ref[...] = xn + jnp.dot(xn, cat2_W_ref[...],
                                preferred_element_type=jnp.float32) + cat2_b_ref[...]


def _dense_body(parts_ref, xn2_ref, w_ref, edge_b_ref, cat1_W_ref, cat1_b_ref,
                f1_W_ref, f1_b_ref, f1_aW_ref, f1_ab_ref, f1_ns_ref, f1_noise_ref,
                f2_W_ref, f2_b_ref, f2_aW_ref, f2_ab_ref, f2_ns_ref, f2_noise_ref,
                out_ref):
    wv = w_ref[...]
    agg = parts_ref[0] + parts_ref[1] + edge_b_ref[...]
    out = agg + jnp.dot(agg, cat1_W_ref[...],
                        preferred_element_type=jnp.float32) + cat1_b_ref[...]
    out = jnp.maximum(out + xn2_ref[...], 0.0)
    out = _style(out, wv, f1_W_ref[...], f1_b_ref[...], f1_aW_ref[...],
                 f1_ab_ref[...], f1_ns_ref[...], f1_noise_ref[...])
    out = _style(out, wv, f2_W_ref[...], f2_b_ref[...], f2_aW_ref[...],
                 f2_ab_ref[...], f2_ns_ref[...], f2_noise_ref[...])
    out_ref[...] = out


def _dense(parts, x, w, edge_b, cat1_W, cat1_b, cat2_W, cat2_b,
           nm_W, nm_b, nm_aW, nm_ab, nm_ns, nm_noise,
           f1_W, f1_b, f1_aW, f1_ab, f1_ns, f1_noise,
           f2_W, f2_b, f2_aW, f2_ab, f2_ns, f2_noise):
    r2 = lambda v: v.reshape(1, -1)
    xn2 = pl.pallas_call(
        _xn_body,
        out_shape=jax.ShapeDtypeStruct((N, H), jnp.float32),
    )(x, w, nm_W, r2(nm_b), nm_aW, r2(nm_ab), r2(nm_ns), nm_noise,
      cat2_W, r2(cat2_b))
    return pl.pallas_call(
        _dense_body,
        out_shape=jax.ShapeDtypeStruct((N, H), jnp.float32),
    )(parts, xn2, w, r2(edge_b), cat1_W, r2(cat1_b),
      f1_W, r2(f1_b), f1_aW, r2(f1_ab), r2(f1_ns), f1_noise,
      f2_W, r2(f2_b), f2_aW, r2(f2_ab), r2(f2_ns), f2_noise)


def kernel(x, edge_index, w, edge_W, edge_b, cat1_W, cat1_b, cat2_W, cat2_b,
           nm_W, nm_b, nm_aW, nm_ab, nm_ns, nm_noise,
           f1_W, f1_b, f1_aW, f1_ab, f1_ns, f1_noise,
           f2_W, f2_b, f2_aW, f2_ab, f2_ns, f2_noise):
    src = edge_index[0]
    dst = edge_index[1]
    npad = EPAD - E
    ar = jnp.arange(npad, dtype=jnp.int32)
    src_p = jnp.concatenate([src, (ar * 131) % N]).reshape(NT * NSUB, CH)
    dst_p = jnp.concatenate([dst, N + (ar % 16)]).reshape(NT * NSUB, CH)
    zeros = jnp.zeros((NA, H), jnp.float32)
    parts = _sc_scatter(src_p, dst_p, edge_W, zeros)
    return _dense(parts, x, w, edge_b, cat1_W, cat1_b, cat2_W, cat2_b,
                  nm_W, nm_b, nm_aW, nm_ab, nm_ns, nm_noise,
                  f1_W, f1_b, f1_aW, f1_ab, f1_ns, f1_noise,
                  f2_W, f2_b, f2_aW, f2_ab, f2_ns, f2_noise)
